# Initial kernel scaffold; baseline (speedup 1.0000x reference)
import functools

import jax
import jax.numpy as jnp
from jax import lax
from jax.experimental import pallas as pl
from jax.experimental.pallas import tpu as pltpu
from jax.experimental.pallas import tpu_sc as plsc

N = 50000
E = 800000
G = 32

NH = N // 2
RP = NH + 8
STRIPE = RP // 16
C = 128
EPS = E // 16
NFULL = EPS // C
TAIL = EPS - NFULL * C
NCHUNK = NFULL + 1
EPAD = E + (NCHUNK * C - EPS)

BLK = 500
NBLK = N // BLK
BPC = NBLK // 2

_HI = jax.lax.Precision.HIGHEST


def _dot(a, b):
    return jnp.dot(a, b, preferred_element_type=jnp.float32, precision=_HI)


def _dg(v, idx):
    return lax.gather(
        v, idx[:, None],
        lax.GatherDimensionNumbers(offset_dims=(), collapsed_slice_dims=(0,),
                                   start_index_map=(0,)),
        (1,), mode=lax.GatherScatterMode.PROMISE_IN_BOUNDS)


def _tc_a_body(x_ref, w1_ref, as_ref, ad_ref, rep_ref,
               h1t_ref, ad1_ref, init1_ref):
    h = _dot(x_ref[...], w1_ref[...])
    a_s = _dot(h, as_ref[...])
    a_d = _dot(h, ad_ref[...])
    al = a_s + a_d
    w = jnp.exp(jnp.maximum(al, 0.2 * al))
    wexp = _dot(w, rep_ref[...])
    h1t_ref[...] = jnp.concatenate([h, a_s], axis=1)
    ad1_ref[...] = a_d
    init1_ref[0] = jnp.concatenate([h * wexp, w], axis=1)


def _tc_a(x, W1, AS, AD, REP):
    return pl.pallas_call(
        _tc_a_body,
        grid=(NBLK,),
        in_specs=[
            pl.BlockSpec((BLK, 512), lambda i: (i, 0)),
            pl.BlockSpec((512, 64), lambda i: (0, 0)),
            pl.BlockSpec((64, 8), lambda i: (0, 0)),
            pl.BlockSpec((64, 8), lambda i: (0, 0)),
            pl.BlockSpec((8, 64), lambda i: (0, 0)),
        ],
        out_specs=[
            pl.BlockSpec((BLK, 72), lambda i: (i, 0)),
            pl.BlockSpec((BLK, 8), lambda i: (i, 0)),
            pl.BlockSpec((1, BLK, 72), lambda i: (i // BPC, i % BPC, 0)),
        ],
        out_shape=[
            jax.ShapeDtypeStruct((N, 72), jnp.float32),
            jax.ShapeDtypeStruct((N, 8), jnp.float32),
            jax.ShapeDtypeStruct((2, RP, 72), jnp.float32),
        ],
    )(x, W1, AS, AD, REP)


def _sc_b_body(srce, dste, h1t, ad1, init1, accum_out,
               src_v, dst_v, idx_v, ad_rows, h_rows, out_rows, accum,
               sem1, sem2):
    c = lax.axis_index("c")
    s = lax.axis_index("s")
    lo = c * NH
    pltpu.sync_copy(init1.at[c, pl.ds(s * STRIPE, STRIPE)],
                    accum.at[pl.ds(s * STRIPE, STRIPE)])
    plsc.subcore_barrier()

    iota = lax.iota(jnp.int32, 16)
    half = iota // 8
    lane8 = iota % 8
    tailcol = 64 + lane8

    def chunk(i, carry):
        goff = s * EPS + i * C
        pltpu.sync_copy(srce.at[pl.ds(goff, C)], src_v)
        pltpu.sync_copy(dste.at[pl.ds(goff, C)], dst_v)
        d1 = pltpu.async_copy(ad1.at[dst_v], ad_rows, sem1)
        d2 = pltpu.async_copy(h1t.at[src_v], h_rows, sem2)
        d1.wait()
        d2.wait()
        for t in range(C // 2):
            rows = 2 * t + half
            asv = plsc.load_gather(h_rows, [rows, tailcol])
            adv = plsc.load_gather(ad_rows, [rows, lane8])
            al = asv + adv
            w = jnp.exp(jnp.maximum(al, 0.2 * al))
            for e, off in ((2 * t, 0), (2 * t + 1, 8)):
                for v in range(4):
                    hv = h_rows[e, pl.ds(16 * v, 16)]
                    wb = _dg(w, off + 2 * v + half)
                    out_rows[e, pl.ds(16 * v, 16)] = hv * wb
            plsc.store_scatter(out_rows, [rows, tailcol], w)
        valid = jnp.where(i < NFULL, C, TAIL)
        for j in range(C // 16):
            d = dst_v[pl.ds(16 * j, 16)]
            keep = (d >= lo) & (d < lo + NH) & ((16 * j + iota) < valid)
            idx_v[pl.ds(16 * j, 16)] = jnp.where(keep, d - lo, NH)
        pltpu.sync_copy(out_rows, accum.at[idx_v], add=True)
        return carry

    lax.fori_loop(0, NCHUNK, chunk, 0)
    plsc.subcore_barrier()
    pltpu.sync_copy(accum.at[pl.ds(s * STRIPE, STRIPE)],
                    accum_out.at[c, pl.ds(s * STRIPE, STRIPE)])


def _sc_b(srce, dste, h1t, ad1, init1):
    mesh = plsc.VectorSubcoreMesh(core_axis_name="c", subcore_axis_name="s",
                                  num_cores=2, num_subcores=16)
    return pl.kernel(
        _sc_b_body,
        out_type=jax.ShapeDtypeStruct((2, RP, 72), jnp.float32),
        mesh=mesh,
        scratch_types=[
            pltpu.VMEM((C,), jnp.int32),
            pltpu.VMEM((C,), jnp.int32),
            pltpu.VMEM((C,), jnp.int32),
            pltpu.VMEM((C, 8), jnp.float32),
            pltpu.VMEM((C, 72), jnp.float32),
            pltpu.VMEM((C, 72), jnp.float32),
            pltpu.VMEM_SHARED((RP, 72), jnp.float32),
            pltpu.SemaphoreType.DMA,
            pltpu.SemaphoreType.DMA,
        ],
    )(srce, dste, h1t, ad1, init1)


def _tc_c_body(acc_ref, rep_ref, b1_ref, w2_ref, as2_ref, ad2_ref,
               h2t_ref, ad2t_ref, init2_ref):
    acc = acc_ref[0]
    num = acc[:, 0:64]
    den = acc[:, 64:72]
    dexp = _dot(den, rep_ref[...])
    o1 = num / dexp + b1_ref[...][None, :]
    o1 = jnp.where(o1 > 0, o1, jnp.exp(o1) - 1.0)
    h2 = _dot(o1, w2_ref[...])
    a_s = _dot(h2, as2_ref[...])
    a_d = _dot(h2, ad2_ref[...])
    al = a_s + a_d
    w = jnp.exp(jnp.maximum(al, 0.2 * al))
    one = jnp.ones((BLK, 1), jnp.float32)
    z14 = jnp.zeros((BLK, 14), jnp.float32)
    z15 = jnp.zeros((BLK, 15), jnp.float32)
    z7 = jnp.zeros((BLK, 7), jnp.float32)
    h2t_ref[...] = jnp.concatenate([h2, one, a_s, z14], axis=1)
    ad2t_ref[...] = jnp.concatenate([a_d, z7], axis=1)
    init2_ref[0] = jnp.concatenate([h2 * w, w, z15], axis=1)


def _tc_c(acc1, REP, b1, W2, as2v, ad2v):
    return pl.pallas_call(
        _tc_c_body,
        grid=(NBLK,),
        in_specs=[
            pl.BlockSpec((1, BLK, 72), lambda i: (i // BPC, i % BPC, 0)),
            pl.BlockSpec((8, 64), lambda i: (0, 0)),
            pl.BlockSpec((64,), lambda i: (0,)),
            pl.BlockSpec((64, 32), lambda i: (0, 0)),
            pl.BlockSpec((32, 1), lambda i: (0, 0)),
            pl.BlockSpec((32, 1), lambda i: (0, 0)),
        ],
        out_specs=[
            pl.BlockSpec((BLK, 48), lambda i: (i, 0)),
            pl.BlockSpec((BLK, 8), lambda i: (i, 0)),
            pl.BlockSpec((1, BLK, 48), lambda i: (i // BPC, i % BPC, 0)),
        ],
        out_shape=[
            jax.ShapeDtypeStruct((N, 48), jnp.float32),
            jax.ShapeDtypeStruct((N, 8), jnp.float32),
            jax.ShapeDtypeStruct((2, RP, 48), jnp.float32),
        ],
    )(acc1, REP, b1, W2, as2v, ad2v)


def _sc_d_body(srce, dste, h2t, ad2t, init2, accum_out,
               src_v, dst_v, idx_v, ad_rows, h_rows, out_rows, accum,
               sem1, sem2):
    c = lax.axis_index("c")
    s = lax.axis_index("s")
    lo = c * NH
    pltpu.sync_copy(init2.at[c, pl.ds(s * STRIPE, STRIPE)],
                    accum.at[pl.ds(s * STRIPE, STRIPE)])
    plsc.subcore_barrier()

    iota = lax.iota(jnp.int32, 16)
    col33 = iota * 0 + 33
    col0 = iota * 0

    def chunk(i, carry):
        goff = s * EPS + i * C
        pltpu.sync_copy(srce.at[pl.ds(goff, C)], src_v)
        pltpu.sync_copy(dste.at[pl.ds(goff, C)], dst_v)
        d1 = pltpu.async_copy(ad2t.at[dst_v], ad_rows, sem1)
        d2 = pltpu.async_copy(h2t.at[src_v], h_rows, sem2)
        d1.wait()
        d2.wait()
        for g in range(C // 16):
            rows = 16 * g + iota
            asv = plsc.load_gather(h_rows, [rows, col33])
            adv = plsc.load_gather(ad_rows, [rows, col0])
            al = asv + adv
            w = jnp.exp(jnp.maximum(al, 0.2 * al))
            for j in range(16):
                e = 16 * g + j
                wb = _dg(w, col0 + j)
                for v in range(3):
                    out_rows[e, pl.ds(16 * v, 16)] = (
                        h_rows[e, pl.ds(16 * v, 16)] * wb)
        valid = jnp.where(i < NFULL, C, TAIL)
        for j in range(C // 16):
            d = dst_v[pl.ds(16 * j, 16)]
            keep = (d >= lo) & (d < lo + NH) & ((16 * j + iota) < valid)
            idx_v[pl.ds(16 * j, 16)] = jnp.where(keep, d - lo, NH)
        pltpu.sync_copy(out_rows, accum.at[idx_v], add=True)
        return carry

    lax.fori_loop(0, NCHUNK, chunk, 0)
    plsc.subcore_barrier()
    pltpu.sync_copy(accum.at[pl.ds(s * STRIPE, STRIPE)],
                    accum_out.at[c, pl.ds(s * STRIPE, STRIPE)])


def _sc_d(srce, dste, h2t, ad2t, init2):
    mesh = plsc.VectorSubcoreMesh(core_axis_name="c", subcore_axis_name="s",
                                  num_cores=2, num_subcores=16)
    return pl.kernel(
        _sc_d_body,
        out_type=jax.ShapeDtypeStruct((2, RP, 48), jnp.float32),
        mesh=mesh,
        scratch_types=[
            pltpu.VMEM((C,), jnp.int32),
            pltpu.VMEM((C,), jnp.int32),
            pltpu.VMEM((C,), jnp.int32),
            pltpu.VMEM((C, 8), jnp.float32),
            pltpu.VMEM((C, 48), jnp.float32),
            pltpu.VMEM((C, 48), jnp.float32),
            pltpu.VMEM_SHARED((RP, 48), jnp.float32),
            pltpu.SemaphoreType.DMA,
            pltpu.SemaphoreType.DMA,
        ],
    )(srce, dste, h2t, ad2t, init2)


def _tc_e_body(acc_ref, batch_ref, b2_ref, linw_ref, linb_ref, out_ref, sacc):
    b = pl.program_id(0)
    acc = acc_ref[0]
    o2 = acc[:, 0:32] / acc[:, 32:33] + b2_ref[...][None, :]
    bb = batch_ref[0]
    gids = lax.broadcasted_iota(jnp.int32, (G, BLK), 0)
    oh = (gids == bb).astype(jnp.float32)
    va = jnp.concatenate([o2, jnp.ones((BLK, 1), jnp.float32)], axis=1)
    part = _dot(oh, va)

    @pl.when(b == 0)
    def _():
        sacc[...] = part

    @pl.when(b > 0)
    def _():
        sacc[...] = sacc[...] + part

    @pl.when(b == NBLK - 1)
    def _():
        tot = sacc[...]
        pooled = tot[:, 0:32] / jnp.maximum(tot[:, 32:33], 1.0)
        out_ref[...] = _dot(pooled, linw_ref[...]) + linb_ref[...][None, :]


def _tc_e(acc2, batch3, b2, linW, linb):
    return pl.pallas_call(
        _tc_e_body,
        grid=(NBLK,),
        in_specs=[
            pl.BlockSpec((1, BLK, 48), lambda i: (i // BPC, i % BPC, 0)),
            pl.BlockSpec((1, 1, BLK), lambda i: (i, 0, 0)),
            pl.BlockSpec((32,), lambda i: (0,)),
            pl.BlockSpec((32, 2), lambda i: (0, 0)),
            pl.BlockSpec((2,), lambda i: (0,)),
        ],
        out_specs=pl.BlockSpec((G, 2), lambda i: (0, 0)),
        out_shape=jax.ShapeDtypeStruct((G, 2), jnp.float32),
        scratch_shapes=[pltpu.VMEM((G, 33), jnp.float32)],
    )(acc2, batch3, b2, linW, linb)


def kernel(x, edge_index, batch, W1, att_src1, att_dst1, b1,
           W2, att_src2, att_dst2, b2, linW, linb):
    pad = jnp.zeros((EPAD - E,), jnp.int32)
    srce = jnp.concatenate([edge_index[0], pad])
    dste = jnp.concatenate([edge_index[1], pad])

    eye8 = jnp.eye(8, dtype=jnp.float32)
    AS = (att_src1[0][:, :, None] * eye8[:, None, :]).reshape(64, 8)
    AD = (att_dst1[0][:, :, None] * eye8[:, None, :]).reshape(64, 8)
    REP = (eye8[:, :, None] * jnp.ones((1, 1, 8), jnp.float32)).reshape(8, 64)
    as2v = att_src2.reshape(32, 1)
    ad2v = att_dst2.reshape(32, 1)

    h1t, ad1, init1 = _tc_a(x, W1, AS, AD, REP)
    acc1 = _sc_b(srce, dste, h1t, ad1, init1)
    h2t, ad2t, init2 = _tc_c(acc1, REP, b1, W2, as2v, ad2v)
    acc2 = _sc_d(srce, dste, h2t, ad2t, init2)
    return _tc_e(acc2, batch.reshape(NBLK, 1, BLK), b2, linW, linb)


# speedup vs baseline: 44.6822x; 44.6822x over previous
import functools

import jax
import jax.numpy as jnp
from jax import lax
from jax.experimental import pallas as pl
from jax.experimental.pallas import tpu as pltpu
from jax.experimental.pallas import tpu_sc as plsc

N = 50000
E = 800000
G = 32

NH = N // 2
RP = NH + 88
STRIPE = RP // 16
C = 128
EPS = E // 16
NFULL = EPS // C
TAIL = EPS - NFULL * C
NCHUNK = NFULL + 1
EPAD = E + (NCHUNK * C - EPS)

BLK = 1000
NBLK = N // BLK
BPC = NBLK // 2

_HI = jax.lax.Precision.HIGHEST


def _dot(a, b):
    return jnp.dot(a, b, preferred_element_type=jnp.float32, precision=_HI)


def _dg(v, idx):
    return lax.gather(
        v, idx[:, None],
        lax.GatherDimensionNumbers(offset_dims=(), collapsed_slice_dims=(0,),
                                   start_index_map=(0,)),
        (1,), mode=lax.GatherScatterMode.PROMISE_IN_BOUNDS)


def _tc_a_body(x_ref, w1_ref, as_ref, ad_ref, rep_ref,
               h1t_ref, ad1_ref, init1_ref):
    h = _dot(x_ref[...], w1_ref[...])
    a_s = _dot(h, as_ref[...])
    a_d = _dot(h, ad_ref[...])
    al = a_s + a_d
    w = jnp.exp(jnp.maximum(al, 0.2 * al))
    wexp = _dot(w, rep_ref[...])
    h1t_ref[...] = jnp.concatenate([h, a_s], axis=1)
    ad1_ref[...] = a_d
    init1_ref[0] = jnp.concatenate([h * wexp, w], axis=1)


def _tc_a(x, W1, AS, AD, REP):
    return pl.pallas_call(
        _tc_a_body,
        grid=(NBLK,),
        in_specs=[
            pl.BlockSpec((BLK, 512), lambda i: (i, 0)),
            pl.BlockSpec((512, 64), lambda i: (0, 0)),
            pl.BlockSpec((64, 8), lambda i: (0, 0)),
            pl.BlockSpec((64, 8), lambda i: (0, 0)),
            pl.BlockSpec((8, 64), lambda i: (0, 0)),
        ],
        out_specs=[
            pl.BlockSpec((BLK, 72), lambda i: (i, 0)),
            pl.BlockSpec((BLK, 8), lambda i: (i, 0)),
            pl.BlockSpec((1, BLK, 72), lambda i: (i // BPC, i % BPC, 0)),
        ],
        out_shape=[
            jax.ShapeDtypeStruct((N, 72), jnp.float32),
            jax.ShapeDtypeStruct((N, 8), jnp.float32),
            jax.ShapeDtypeStruct((2, RP, 72), jnp.float32),
        ],
    )(x, W1, AS, AD, REP)


def _sc_b_body(srce, dste, h1t, ad1, init1, accum_out,
               src_v, dst_v, idx_v, ad_rows, h_rows, accum,
               sem1, sem2):
    c = lax.axis_index("c")
    s = lax.axis_index("s")
    lo = c * NH
    pltpu.sync_copy(init1.at[c, pl.ds(s * STRIPE, STRIPE)],
                    accum.at[pl.ds(s * STRIPE, STRIPE)])
    plsc.subcore_barrier()

    iota = lax.iota(jnp.int32, 16)
    half = iota // 8
    lane8 = iota % 8
    tailcol = 64 + lane8

    def chunk(i, carry):
        goff = s * EPS + i * C
        pltpu.sync_copy(srce.at[pl.ds(goff, C)], src_v)
        pltpu.sync_copy(dste.at[pl.ds(goff, C)], dst_v)
        d1 = pltpu.async_copy(ad1.at[dst_v], ad_rows, sem1)
        d2 = pltpu.async_copy(h1t.at[src_v], h_rows, sem2)
        d1.wait()
        d2.wait()
        for t in range(C // 2):
            rows = 2 * t + half
            asv = plsc.load_gather(h_rows, [rows, tailcol])
            adv = plsc.load_gather(ad_rows, [rows, lane8])
            al = asv + adv
            w = jnp.exp(jnp.maximum(al, 0.2 * al))
            for e, off in ((2 * t, 0), (2 * t + 1, 8)):
                for v in range(4):
                    hv = h_rows[e, pl.ds(16 * v, 16)]
                    wb = _dg(w, off + 2 * v + half)
                    h_rows[e, pl.ds(16 * v, 16)] = hv * wb
            plsc.store_scatter(h_rows, [rows, tailcol], w)
        valid = jnp.where(i < NFULL, C, TAIL)
        for j in range(C // 16):
            d = dst_v[pl.ds(16 * j, 16)]
            keep = (d >= lo) & (d < lo + NH) & ((16 * j + iota) < valid)
            idx_v[pl.ds(16 * j, 16)] = jnp.where(keep, d - lo, NH)
        pltpu.sync_copy(h_rows, accum.at[idx_v], add=True)
        return carry

    lax.fori_loop(0, NCHUNK, chunk, 0)
    plsc.subcore_barrier()
    pltpu.sync_copy(accum.at[pl.ds(s * STRIPE, STRIPE)],
                    accum_out.at[c, pl.ds(s * STRIPE, STRIPE)])


def _sc_b(srce, dste, h1t, ad1, init1):
    mesh = plsc.VectorSubcoreMesh(core_axis_name="c", subcore_axis_name="s",
                                  num_cores=2, num_subcores=16)
    return pl.kernel(
        _sc_b_body,
        out_type=jax.ShapeDtypeStruct((2, RP, 72), jnp.float32),
        mesh=mesh,
        compiler_params=pltpu.CompilerParams(needs_layout_passes=False, use_tc_tiling_on_sc=False),
        scratch_types=[
            pltpu.VMEM((C,), jnp.int32),
            pltpu.VMEM((C,), jnp.int32),
            pltpu.VMEM((C,), jnp.int32),
            pltpu.VMEM((C, 8), jnp.float32),
            pltpu.VMEM((C, 72), jnp.float32),
            pltpu.VMEM_SHARED((RP, 72), jnp.float32),
            pltpu.SemaphoreType.DMA,
            pltpu.SemaphoreType.DMA,
        ],
    )(srce, dste, h1t, ad1, init1)


def _tc_c_body(acc_ref, rep_ref, b1_ref, w2_ref, as2_ref, ad2_ref,
               h2t_ref, ad2t_ref, init2_ref):
    acc = acc_ref[0]
    num = acc[:, 0:64]
    den = acc[:, 64:72]
    dexp = _dot(den, rep_ref[...])
    o1 = num / dexp + b1_ref[...][None, :]
    o1 = jnp.where(o1 > 0, o1, jnp.exp(o1) - 1.0)
    h2 = _dot(o1, w2_ref[...])
    a_s = _dot(h2, as2_ref[...])
    a_d = _dot(h2, ad2_ref[...])
    al = a_s + a_d
    w = jnp.exp(jnp.maximum(al, 0.2 * al))
    one = jnp.ones((BLK, 1), jnp.float32)
    z14 = jnp.zeros((BLK, 14), jnp.float32)
    z15 = jnp.zeros((BLK, 15), jnp.float32)
    z7 = jnp.zeros((BLK, 7), jnp.float32)
    h2t_ref[...] = jnp.concatenate([h2, one, a_s, z14], axis=1)
    ad2t_ref[...] = jnp.concatenate([a_d, z7], axis=1)
    init2_ref[0] = jnp.concatenate([h2 * w, w, z15], axis=1)


def _tc_c(acc1, REP, b1, W2, as2v, ad2v):
    return pl.pallas_call(
        _tc_c_body,
        grid=(NBLK,),
        in_specs=[
            pl.BlockSpec((1, BLK, 72), lambda i: (i // BPC, i % BPC, 0)),
            pl.BlockSpec((8, 64), lambda i: (0, 0)),
            pl.BlockSpec((64,), lambda i: (0,)),
            pl.BlockSpec((64, 32), lambda i: (0, 0)),
            pl.BlockSpec((32, 1), lambda i: (0, 0)),
            pl.BlockSpec((32, 1), lambda i: (0, 0)),
        ],
        out_specs=[
            pl.BlockSpec((BLK, 48), lambda i: (i, 0)),
            pl.BlockSpec((BLK, 8), lambda i: (i, 0)),
            pl.BlockSpec((1, BLK, 48), lambda i: (i // BPC, i % BPC, 0)),
        ],
        out_shape=[
            jax.ShapeDtypeStruct((N, 48), jnp.float32),
            jax.ShapeDtypeStruct((N, 8), jnp.float32),
            jax.ShapeDtypeStruct((2, RP, 48), jnp.float32),
        ],
    )(acc1, REP, b1, W2, as2v, ad2v)


def _sc_d_body(srce, dste, h2t, ad2t, init2, accum_out,
               src_v, dst_v, idx_v, ad_rows, h_rows, accum,
               sem1, sem2):
    c = lax.axis_index("c")
    s = lax.axis_index("s")
    lo = c * NH
    pltpu.sync_copy(init2.at[c, pl.ds(s * STRIPE, STRIPE)],
                    accum.at[pl.ds(s * STRIPE, STRIPE)])
    plsc.subcore_barrier()

    iota = lax.iota(jnp.int32, 16)
    col33 = iota * 0 + 33
    col0 = iota * 0

    def chunk(i, carry):
        goff = s * EPS + i * C
        pltpu.sync_copy(srce.at[pl.ds(goff, C)], src_v)
        pltpu.sync_copy(dste.at[pl.ds(goff, C)], dst_v)
        d1 = pltpu.async_copy(ad2t.at[dst_v], ad_rows, sem1)
        d2 = pltpu.async_copy(h2t.at[src_v], h_rows, sem2)
        d1.wait()
        d2.wait()
        for g in range(C // 16):
            rows = 16 * g + iota
            asv = plsc.load_gather(h_rows, [rows, col33])
            adv = plsc.load_gather(ad_rows, [rows, col0])
            al = asv + adv
            w = jnp.exp(jnp.maximum(al, 0.2 * al))
            for j in range(16):
                e = 16 * g + j
                wb = _dg(w, col0 + j)
                for v in range(3):
                    h_rows[e, pl.ds(16 * v, 16)] = (
                        h_rows[e, pl.ds(16 * v, 16)] * wb)
        valid = jnp.where(i < NFULL, C, TAIL)
        for j in range(C // 16):
            d = dst_v[pl.ds(16 * j, 16)]
            keep = (d >= lo) & (d < lo + NH) & ((16 * j + iota) < valid)
            idx_v[pl.ds(16 * j, 16)] = jnp.where(keep, d - lo, NH)
        pltpu.sync_copy(h_rows, accum.at[idx_v], add=True)
        return carry

    lax.fori_loop(0, NCHUNK, chunk, 0)
    plsc.subcore_barrier()
    pltpu.sync_copy(accum.at[pl.ds(s * STRIPE, STRIPE)],
                    accum_out.at[c, pl.ds(s * STRIPE, STRIPE)])


def _sc_d(srce, dste, h2t, ad2t, init2):
    mesh = plsc.VectorSubcoreMesh(core_axis_name="c", subcore_axis_name="s",
                                  num_cores=2, num_subcores=16)
    return pl.kernel(
        _sc_d_body,
        out_type=jax.ShapeDtypeStruct((2, RP, 48), jnp.float32),
        mesh=mesh,
        compiler_params=pltpu.CompilerParams(needs_layout_passes=False, use_tc_tiling_on_sc=False),
        scratch_types=[
            pltpu.VMEM((C,), jnp.int32),
            pltpu.VMEM((C,), jnp.int32),
            pltpu.VMEM((C,), jnp.int32),
            pltpu.VMEM((C, 8), jnp.float32),
            pltpu.VMEM((C, 48), jnp.float32),
            pltpu.VMEM_SHARED((RP, 48), jnp.float32),
            pltpu.SemaphoreType.DMA,
            pltpu.SemaphoreType.DMA,
        ],
    )(srce, dste, h2t, ad2t, init2)


def _tc_e_body(acc_ref, batch_ref, b2_ref, linw_ref, linb_ref, out_ref, sacc):
    b = pl.program_id(0)
    acc = acc_ref[0]
    o2 = acc[:, 0:32] / acc[:, 32:33] + b2_ref[...][None, :]
    bb = batch_ref[0]
    gids = lax.broadcasted_iota(jnp.int32, (G, BLK), 0)
    oh = (gids == bb).astype(jnp.float32)
    va = jnp.concatenate([o2, jnp.ones((BLK, 1), jnp.float32)], axis=1)
    part = _dot(oh, va)

    @pl.when(b == 0)
    def _():
        sacc[...] = part

    @pl.when(b > 0)
    def _():
        sacc[...] = sacc[...] + part

    @pl.when(b == NBLK - 1)
    def _():
        tot = sacc[...]
        pooled = tot[:, 0:32] / jnp.maximum(tot[:, 32:33], 1.0)
        out_ref[...] = _dot(pooled, linw_ref[...]) + linb_ref[...][None, :]


def _tc_e(acc2, batch3, b2, linW, linb):
    return pl.pallas_call(
        _tc_e_body,
        grid=(NBLK,),
        in_specs=[
            pl.BlockSpec((1, BLK, 48), lambda i: (i // BPC, i % BPC, 0)),
            pl.BlockSpec((1, 1, BLK), lambda i: (i, 0, 0)),
            pl.BlockSpec((32,), lambda i: (0,)),
            pl.BlockSpec((32, 2), lambda i: (0, 0)),
            pl.BlockSpec((2,), lambda i: (0,)),
        ],
        out_specs=pl.BlockSpec((G, 2), lambda i: (0, 0)),
        out_shape=jax.ShapeDtypeStruct((G, 2), jnp.float32),
        scratch_shapes=[pltpu.VMEM((G, 33), jnp.float32)],
    )(acc2, batch3, b2, linW, linb)


def kernel(x, edge_index, batch, W1, att_src1, att_dst1, b1,
           W2, att_src2, att_dst2, b2, linW, linb):
    pad = jnp.zeros((EPAD - E,), jnp.int32)
    srce = jnp.concatenate([edge_index[0], pad])
    dste = jnp.concatenate([edge_index[1], pad])

    eye8 = jnp.eye(8, dtype=jnp.float32)
    AS = (att_src1[0][:, :, None] * eye8[:, None, :]).reshape(64, 8)
    AD = (att_dst1[0][:, :, None] * eye8[:, None, :]).reshape(64, 8)
    REP = (eye8[:, :, None] * jnp.ones((1, 1, 8), jnp.float32)).reshape(8, 64)
    as2v = att_src2.reshape(32, 1)
    ad2v = att_dst2.reshape(32, 1)

    h1t, ad1, init1 = _tc_a(x, W1, AS, AD, REP)
    acc1 = _sc_b(srce, dste, h1t, ad1, init1)
    h2t, ad2t, init2 = _tc_c(acc1, REP, b1, W2, as2v, ad2v)
    acc2 = _sc_d(srce, dste, h2t, ad2t, init2)
    return _tc_e(acc2, batch.reshape(NBLK, 1, BLK), b2, linW, linb)


# speedup vs baseline: 74.7383x; 1.6727x over previous
import functools

import jax
import jax.numpy as jnp
from jax import lax
from jax.experimental import pallas as pl
from jax.experimental.pallas import tpu as pltpu
from jax.experimental.pallas import tpu_sc as plsc

N = 50000
E = 800000
G = 32

NH = N // 2
RP = NH + 88
STRIPE = RP // 16
C = 96
EPS = E // 16
NCHUNK = 522
EPAD = 15 * EPS + (NCHUNK + 2) * C

BLK = 1000
NBLK = N // BLK
BPC = NBLK // 2

_HI = jax.lax.Precision.HIGHEST


def _dot(a, b):
    return jnp.dot(a, b, preferred_element_type=jnp.float32, precision=_HI)


def _dg(v, idx):
    return lax.gather(
        v, idx[:, None],
        lax.GatherDimensionNumbers(offset_dims=(), collapsed_slice_dims=(0,),
                                   start_index_map=(0,)),
        (1,), mode=lax.GatherScatterMode.PROMISE_IN_BOUNDS)


def _tc_a_body(x_ref, w1_ref, as_ref, ad_ref, rep_ref,
               h1t_ref, ad1_ref, init1_ref):
    h = _dot(x_ref[...], w1_ref[...])
    a_s = _dot(h, as_ref[...])
    a_d = _dot(h, ad_ref[...])
    al = a_s + a_d
    w = jnp.exp(jnp.maximum(al, 0.2 * al))
    wexp = _dot(w, rep_ref[...])
    h1t_ref[...] = jnp.concatenate([h, a_s], axis=1)
    ad1_ref[...] = a_d
    init1_ref[0] = jnp.concatenate([h * wexp, w], axis=1)


def _tc_a(x, W1, AS, AD, REP):
    return pl.pallas_call(
        _tc_a_body,
        grid=(NBLK,),
        in_specs=[
            pl.BlockSpec((BLK, 512), lambda i: (i, 0)),
            pl.BlockSpec((512, 64), lambda i: (0, 0)),
            pl.BlockSpec((64, 8), lambda i: (0, 0)),
            pl.BlockSpec((64, 8), lambda i: (0, 0)),
            pl.BlockSpec((8, 64), lambda i: (0, 0)),
        ],
        out_specs=[
            pl.BlockSpec((BLK, 72), lambda i: (i, 0)),
            pl.BlockSpec((BLK, 8), lambda i: (i, 0)),
            pl.BlockSpec((1, BLK, 72), lambda i: (i // BPC, i % BPC, 0)),
        ],
        out_shape=[
            jax.ShapeDtypeStruct((N, 72), jnp.float32),
            jax.ShapeDtypeStruct((N, 8), jnp.float32),
            jax.ShapeDtypeStruct((2, RP, 72), jnp.float32),
        ],
    )(x, W1, AS, AD, REP)


def _sc_b_body(srce, dste, h1t, ad1, init1, accum_out,
               src0, dst0, idx0, ad0, h0, sl0, sg0,
               src1, dst1, idx1, ad1r, h1r, sl1, sg1, accum):
    c = lax.axis_index("c")
    s = lax.axis_index("s")
    lo = c * NH
    pltpu.sync_copy(init1.at[c, pl.ds(s * STRIPE, STRIPE)],
                    accum.at[pl.ds(s * STRIPE, STRIPE)])
    plsc.subcore_barrier()
    iota = lax.iota(jnp.int32, 16)
    half = iota // 8
    lane8 = iota % 8
    tailcol = 64 + lane8
    bufs = ((src0, dst0, idx0, ad0, h0, sl0, sg0),
            (src1, dst1, idx1, ad1r, h1r, sl1, sg1))

    def issue_loads(i, b):
        goff = s * EPS + i * C
        pltpu.async_copy(srce.at[pl.ds(goff, C)], bufs[b][0], bufs[b][5])
        pltpu.async_copy(dste.at[pl.ds(goff, C)], bufs[b][1], bufs[b][5])

    def wait_loads(b):
        pltpu.make_async_copy(srce.at[pl.ds(0, C)], bufs[b][0], bufs[b][5]).wait()
        pltpu.make_async_copy(dste.at[pl.ds(0, C)], bufs[b][1], bufs[b][5]).wait()

    def issue_gathers(b):
        pltpu.async_copy(ad1.at[bufs[b][1]], bufs[b][3], bufs[b][6])
        pltpu.async_copy(h1t.at[bufs[b][0]], bufs[b][4], bufs[b][6])

    def wait_gathers(b):
        pltpu.make_async_copy(ad1.at[pl.ds(0, C)], bufs[b][3], bufs[b][6]).wait()
        pltpu.make_async_copy(h1t.at[pl.ds(0, C)], bufs[b][4], bufs[b][6]).wait()

    def phase(i, b):
        o = 1 - b
        wait_loads(o)
        issue_gathers(o)
        dv, xv = bufs[b][1], bufs[b][2]
        valid = jnp.minimum(jnp.maximum(EPS - i * C, 0), C)
        for j in range(C // 16):
            d = dv[pl.ds(16 * j, 16)]
            keep = (d >= lo) & (d < lo + NH) & ((16 * j + iota) < valid)
            xv[pl.ds(16 * j, 16)] = jnp.where(keep, d - lo, NH)
        wait_gathers(b)
        issue_loads(i + 2, b)
        hr, ar = bufs[b][4], bufs[b][3]

        def group(g, carry):
            for u in range(4):
                t = 4 * g + u
                rows = 2 * t + half
                asv = plsc.load_gather(hr, [rows, tailcol])
                adv = plsc.load_gather(ar, [rows, lane8])
                al = asv + adv
                w = jnp.exp(jnp.maximum(al, 0.2 * al))
                for eo, off in ((0, 0), (1, 8)):
                    e = 2 * t + eo
                    for v in range(4):
                        hv = hr[e, pl.ds(16 * v, 16)]
                        wb = _dg(w, off + 2 * v + half)
                        hr[e, pl.ds(16 * v, 16)] = hv * wb
                plsc.store_scatter(hr, [rows, tailcol], w)
            return carry

        lax.fori_loop(0, C // 8, group, 0)
        pltpu.sync_copy(hr, accum.at[xv], add=True)

    pltpu.sync_copy(srce.at[pl.ds(s * EPS, C)], src0)
    pltpu.sync_copy(dste.at[pl.ds(s * EPS, C)], dst0)
    issue_gathers(0)
    issue_loads(1, 1)

    def pair(k, carry):
        phase(2 * k, 0)
        phase(2 * k + 1, 1)
        return carry

    lax.fori_loop(0, NCHUNK // 2, pair, 0)
    wait_gathers(0)
    wait_loads(1)
    plsc.subcore_barrier()
    pltpu.sync_copy(accum.at[pl.ds(s * STRIPE, STRIPE)],
                    accum_out.at[c, pl.ds(s * STRIPE, STRIPE)])


def _sc_b(srce, dste, h1t, ad1, init1):
    mesh = plsc.VectorSubcoreMesh(core_axis_name="c", subcore_axis_name="s",
                                  num_cores=2, num_subcores=16)
    return pl.kernel(
        _sc_b_body,
        out_type=jax.ShapeDtypeStruct((2, RP, 72), jnp.float32),
        mesh=mesh,
        compiler_params=pltpu.CompilerParams(needs_layout_passes=False, use_tc_tiling_on_sc=False),
        scratch_types=[
            pltpu.VMEM((C,), jnp.int32),
            pltpu.VMEM((C,), jnp.int32),
            pltpu.VMEM((C,), jnp.int32),
            pltpu.VMEM((C, 8), jnp.float32),
            pltpu.VMEM((C, 72), jnp.float32),
            pltpu.SemaphoreType.DMA,
            pltpu.SemaphoreType.DMA,
            pltpu.VMEM((C,), jnp.int32),
            pltpu.VMEM((C,), jnp.int32),
            pltpu.VMEM((C,), jnp.int32),
            pltpu.VMEM((C, 8), jnp.float32),
            pltpu.VMEM((C, 72), jnp.float32),
            pltpu.SemaphoreType.DMA,
            pltpu.SemaphoreType.DMA,
            pltpu.VMEM_SHARED((RP, 72), jnp.float32),
        ],
    )(srce, dste, h1t, ad1, init1)


def _tc_c_body(acc_ref, rep_ref, b1_ref, w2_ref, as2_ref, ad2_ref,
               h2t_ref, ad2t_ref, init2_ref):
    acc = acc_ref[0]
    num = acc[:, 0:64]
    den = acc[:, 64:72]
    dexp = _dot(den, rep_ref[...])
    o1 = num / dexp + b1_ref[...][None, :]
    o1 = jnp.where(o1 > 0, o1, jnp.exp(o1) - 1.0)
    h2 = _dot(o1, w2_ref[...])
    a_s = _dot(h2, as2_ref[...])
    a_d = _dot(h2, ad2_ref[...])
    al = a_s + a_d
    w = jnp.exp(jnp.maximum(al, 0.2 * al))
    one = jnp.ones((BLK, 1), jnp.float32)
    z14 = jnp.zeros((BLK, 14), jnp.float32)
    z15 = jnp.zeros((BLK, 15), jnp.float32)
    z7 = jnp.zeros((BLK, 7), jnp.float32)
    h2t_ref[...] = jnp.concatenate([h2, one, a_s, z14], axis=1)
    ad2t_ref[...] = jnp.concatenate([a_d, z7], axis=1)
    init2_ref[0] = jnp.concatenate([h2 * w, w, z15], axis=1)


def _tc_c(acc1, REP, b1, W2, as2v, ad2v):
    return pl.pallas_call(
        _tc_c_body,
        grid=(NBLK,),
        in_specs=[
            pl.BlockSpec((1, BLK, 72), lambda i: (i // BPC, i % BPC, 0)),
            pl.BlockSpec((8, 64), lambda i: (0, 0)),
            pl.BlockSpec((64,), lambda i: (0,)),
            pl.BlockSpec((64, 32), lambda i: (0, 0)),
            pl.BlockSpec((32, 1), lambda i: (0, 0)),
            pl.BlockSpec((32, 1), lambda i: (0, 0)),
        ],
        out_specs=[
            pl.BlockSpec((BLK, 48), lambda i: (i, 0)),
            pl.BlockSpec((BLK, 8), lambda i: (i, 0)),
            pl.BlockSpec((1, BLK, 48), lambda i: (i // BPC, i % BPC, 0)),
        ],
        out_shape=[
            jax.ShapeDtypeStruct((N, 48), jnp.float32),
            jax.ShapeDtypeStruct((N, 8), jnp.float32),
            jax.ShapeDtypeStruct((2, RP, 48), jnp.float32),
        ],
    )(acc1, REP, b1, W2, as2v, ad2v)


def _sc_d_body(srce, dste, h2t, ad2t, init2, accum_out,
               src0, dst0, idx0, ad0, h0, sl0, sg0,
               src1, dst1, idx1, adb1, hb1, sl1, sg1, accum):
    c = lax.axis_index("c")
    s = lax.axis_index("s")
    lo = c * NH
    pltpu.sync_copy(init2.at[c, pl.ds(s * STRIPE, STRIPE)],
                    accum.at[pl.ds(s * STRIPE, STRIPE)])
    plsc.subcore_barrier()
    iota = lax.iota(jnp.int32, 16)
    col33 = iota * 0 + 33
    col0 = iota * 0
    bufs = ((src0, dst0, idx0, ad0, h0, sl0, sg0),
            (src1, dst1, idx1, adb1, hb1, sl1, sg1))

    def issue_loads(i, b):
        goff = s * EPS + i * C
        pltpu.async_copy(srce.at[pl.ds(goff, C)], bufs[b][0], bufs[b][5])
        pltpu.async_copy(dste.at[pl.ds(goff, C)], bufs[b][1], bufs[b][5])

    def wait_loads(b):
        pltpu.make_async_copy(srce.at[pl.ds(0, C)], bufs[b][0], bufs[b][5]).wait()
        pltpu.make_async_copy(dste.at[pl.ds(0, C)], bufs[b][1], bufs[b][5]).wait()

    def issue_gathers(b):
        pltpu.async_copy(ad2t.at[bufs[b][1]], bufs[b][3], bufs[b][6])
        pltpu.async_copy(h2t.at[bufs[b][0]], bufs[b][4], bufs[b][6])

    def wait_gathers(b):
        pltpu.make_async_copy(ad2t.at[pl.ds(0, C)], bufs[b][3], bufs[b][6]).wait()
        pltpu.make_async_copy(h2t.at[pl.ds(0, C)], bufs[b][4], bufs[b][6]).wait()

    def phase(i, b):
        o = 1 - b
        wait_loads(o)
        issue_gathers(o)
        dv, xv = bufs[b][1], bufs[b][2]
        valid = jnp.minimum(jnp.maximum(EPS - i * C, 0), C)
        for j in range(C // 16):
            d = dv[pl.ds(16 * j, 16)]
            keep = (d >= lo) & (d < lo + NH) & ((16 * j + iota) < valid)
            xv[pl.ds(16 * j, 16)] = jnp.where(keep, d - lo, NH)
        wait_gathers(b)
        issue_loads(i + 2, b)
        hr, ar = bufs[b][4], bufs[b][3]

        def group(g, carry):
            base = 16 * g
            rows = base + iota
            asv = plsc.load_gather(hr, [rows, col33])
            adv = plsc.load_gather(ar, [rows, col0])
            al = asv + adv
            w = jnp.exp(jnp.maximum(al, 0.2 * al))
            for j in range(16):
                e = base + j
                wb = _dg(w, col0 + j)
                for v in range(3):
                    hr[e, pl.ds(16 * v, 16)] = hr[e, pl.ds(16 * v, 16)] * wb
            return carry

        lax.fori_loop(0, C // 16, group, 0)
        pltpu.sync_copy(hr, accum.at[xv], add=True)

    pltpu.sync_copy(srce.at[pl.ds(s * EPS, C)], src0)
    pltpu.sync_copy(dste.at[pl.ds(s * EPS, C)], dst0)
    issue_gathers(0)
    issue_loads(1, 1)

    def pair(k, carry):
        phase(2 * k, 0)
        phase(2 * k + 1, 1)
        return carry

    lax.fori_loop(0, NCHUNK // 2, pair, 0)
    wait_gathers(0)
    wait_loads(1)
    plsc.subcore_barrier()
    pltpu.sync_copy(accum.at[pl.ds(s * STRIPE, STRIPE)],
                    accum_out.at[c, pl.ds(s * STRIPE, STRIPE)])


def _sc_d(srce, dste, h2t, ad2t, init2):
    mesh = plsc.VectorSubcoreMesh(core_axis_name="c", subcore_axis_name="s",
                                  num_cores=2, num_subcores=16)
    return pl.kernel(
        _sc_d_body,
        out_type=jax.ShapeDtypeStruct((2, RP, 48), jnp.float32),
        mesh=mesh,
        compiler_params=pltpu.CompilerParams(needs_layout_passes=False, use_tc_tiling_on_sc=False),
        scratch_types=[
            pltpu.VMEM((C,), jnp.int32),
            pltpu.VMEM((C,), jnp.int32),
            pltpu.VMEM((C,), jnp.int32),
            pltpu.VMEM((C, 8), jnp.float32),
            pltpu.VMEM((C, 48), jnp.float32),
            pltpu.SemaphoreType.DMA,
            pltpu.SemaphoreType.DMA,
            pltpu.VMEM((C,), jnp.int32),
            pltpu.VMEM((C,), jnp.int32),
            pltpu.VMEM((C,), jnp.int32),
            pltpu.VMEM((C, 8), jnp.float32),
            pltpu.VMEM((C, 48), jnp.float32),
            pltpu.SemaphoreType.DMA,
            pltpu.SemaphoreType.DMA,
            pltpu.VMEM_SHARED((RP, 48), jnp.float32),
        ],
    )(srce, dste, h2t, ad2t, init2)


def _tc_e_body(acc_ref, batch_ref, b2_ref, linw_ref, linb_ref, out_ref, sacc):
    b = pl.program_id(0)
    acc = acc_ref[0]
    o2 = acc[:, 0:32] / acc[:, 32:33] + b2_ref[...][None, :]
    bb = batch_ref[0]
    gids = lax.broadcasted_iota(jnp.int32, (G, BLK), 0)
    oh = (gids == bb).astype(jnp.float32)
    va = jnp.concatenate([o2, jnp.ones((BLK, 1), jnp.float32)], axis=1)
    part = _dot(oh, va)

    @pl.when(b == 0)
    def _():
        sacc[...] = part

    @pl.when(b > 0)
    def _():
        sacc[...] = sacc[...] + part

    @pl.when(b == NBLK - 1)
    def _():
        tot = sacc[...]
        pooled = tot[:, 0:32] / jnp.maximum(tot[:, 32:33], 1.0)
        out_ref[...] = _dot(pooled, linw_ref[...]) + linb_ref[...][None, :]


def _tc_e(acc2, batch3, b2, linW, linb):
    return pl.pallas_call(
        _tc_e_body,
        grid=(NBLK,),
        in_specs=[
            pl.BlockSpec((1, BLK, 48), lambda i: (i // BPC, i % BPC, 0)),
            pl.BlockSpec((1, 1, BLK), lambda i: (i, 0, 0)),
            pl.BlockSpec((32,), lambda i: (0,)),
            pl.BlockSpec((32, 2), lambda i: (0, 0)),
            pl.BlockSpec((2,), lambda i: (0,)),
        ],
        out_specs=pl.BlockSpec((G, 2), lambda i: (0, 0)),
        out_shape=jax.ShapeDtypeStruct((G, 2), jnp.float32),
        scratch_shapes=[pltpu.VMEM((G, 33), jnp.float32)],
    )(acc2, batch3, b2, linW, linb)


def kernel(x, edge_index, batch, W1, att_src1, att_dst1, b1,
           W2, att_src2, att_dst2, b2, linW, linb):
    pad = jnp.zeros((EPAD - E,), jnp.int32)
    srce = jnp.concatenate([edge_index[0], pad])
    dste = jnp.concatenate([edge_index[1], pad])

    eye8 = jnp.eye(8, dtype=jnp.float32)
    AS = (att_src1[0][:, :, None] * eye8[:, None, :]).reshape(64, 8)
    AD = (att_dst1[0][:, :, None] * eye8[:, None, :]).reshape(64, 8)
    REP = (eye8[:, :, None] * jnp.ones((1, 1, 8), jnp.float32)).reshape(8, 64)
    as2v = att_src2.reshape(32, 1)
    ad2v = att_dst2.reshape(32, 1)

    h1t, ad1, init1 = _tc_a(x, W1, AS, AD, REP)
    acc1 = _sc_b(srce, dste, h1t, ad1, init1)
    h2t, ad2t, init2 = _tc_c(acc1, REP, b1, W2, as2v, ad2v)
    acc2 = _sc_d(srce, dste, h2t, ad2t, init2)
    return _tc_e(acc2, batch.reshape(NBLK, 1, BLK), b2, linW, linb)


# speedup vs baseline: 95.0793x; 1.2722x over previous
import functools

import jax
import jax.numpy as jnp
from jax import lax
from jax.experimental import pallas as pl
from jax.experimental.pallas import tpu as pltpu
from jax.experimental.pallas import tpu_sc as plsc

N = 50000
E = 800000
G = 32

NH = N // 2
RP = NH + 88
STRIPE = RP // 16
C = 96
EPS = E // 16
NCHUNK = 522
EPAD = 15 * EPS + (NCHUNK + 2) * C

BLK = 1000
NBLK = N // BLK
BPC = NBLK // 2

_HI = jax.lax.Precision.HIGHEST


def _dot(a, b, prec=None):
    return jnp.dot(a, b, preferred_element_type=jnp.float32, precision=prec)


def _dg(v, idx):
    return lax.gather(
        v, idx[:, None],
        lax.GatherDimensionNumbers(offset_dims=(), collapsed_slice_dims=(0,),
                                   start_index_map=(0,)),
        (1,), mode=lax.GatherScatterMode.PROMISE_IN_BOUNDS)


def _tc_a_body(x_ref, w1_ref, sad_ref, rep_ref,
               h1t_ref, ad1_ref, init1_ref):
    h = _dot(x_ref[...], w1_ref[...])
    asad = _dot(h, sad_ref[...])
    a_s = asad[:, 0:8]
    a_d = asad[:, 8:16]
    al = a_s + a_d
    w = jnp.exp(jnp.maximum(al, 0.2 * al))
    wexp = _dot(w, rep_ref[...])
    h1t_ref[...] = jnp.concatenate([h, a_s], axis=1)
    ad1_ref[...] = a_d
    init1_ref[0] = jnp.concatenate([h * wexp, w], axis=1)


def _tc_a(x, W1, SAD, REP):
    return pl.pallas_call(
        _tc_a_body,
        grid=(NBLK,),
        in_specs=[
            pl.BlockSpec((BLK, 512), lambda i: (i, 0)),
            pl.BlockSpec((512, 64), lambda i: (0, 0)),
            pl.BlockSpec((64, 16), lambda i: (0, 0)),
            pl.BlockSpec((8, 64), lambda i: (0, 0)),
        ],
        out_specs=[
            pl.BlockSpec((BLK, 72), lambda i: (i, 0)),
            pl.BlockSpec((BLK, 8), lambda i: (i, 0)),
            pl.BlockSpec((1, BLK, 72), lambda i: (i // BPC, i % BPC, 0)),
        ],
        out_shape=[
            jax.ShapeDtypeStruct((N, 72), jnp.float32),
            jax.ShapeDtypeStruct((N, 8), jnp.float32),
            jax.ShapeDtypeStruct((2, RP, 72), jnp.float32),
        ],
    )(x, W1, SAD, REP)


def _sc_b_body(srce, dste, h1t, ad1, init1, accum_out,
               src0, dst0, idx0, ad0, h0, sl0, sg0,
               src1, dst1, idx1, ad1r, h1r, sl1, sg1, accum):
    c = lax.axis_index("c")
    s = lax.axis_index("s")
    lo = c * NH
    pltpu.sync_copy(init1.at[c, pl.ds(s * STRIPE, STRIPE)],
                    accum.at[pl.ds(s * STRIPE, STRIPE)])
    plsc.subcore_barrier()
    iota = lax.iota(jnp.int32, 16)
    half = iota // 8
    lane8 = iota % 8
    tailcol = 64 + lane8
    bufs = ((src0, dst0, idx0, ad0, h0, sl0, sg0),
            (src1, dst1, idx1, ad1r, h1r, sl1, sg1))

    def issue_loads(i, b):
        goff = s * EPS + i * C
        pltpu.async_copy(srce.at[pl.ds(goff, C)], bufs[b][0], bufs[b][5])
        pltpu.async_copy(dste.at[pl.ds(goff, C)], bufs[b][1], bufs[b][5])

    def wait_loads(b):
        pltpu.make_async_copy(srce.at[pl.ds(0, C)], bufs[b][0], bufs[b][5]).wait()
        pltpu.make_async_copy(dste.at[pl.ds(0, C)], bufs[b][1], bufs[b][5]).wait()

    def issue_gathers(b):
        pltpu.async_copy(ad1.at[bufs[b][1]], bufs[b][3], bufs[b][6])
        pltpu.async_copy(h1t.at[bufs[b][0]], bufs[b][4], bufs[b][6])

    def wait_gathers(b):
        pltpu.make_async_copy(ad1.at[pl.ds(0, C)], bufs[b][3], bufs[b][6]).wait()
        pltpu.make_async_copy(h1t.at[pl.ds(0, C)], bufs[b][4], bufs[b][6]).wait()

    def phase(i, b):
        o = 1 - b
        wait_loads(o)
        issue_gathers(o)
        dv, xv = bufs[b][1], bufs[b][2]
        valid = jnp.minimum(jnp.maximum(EPS - i * C, 0), C)
        for j in range(C // 16):
            d = dv[pl.ds(16 * j, 16)]
            keep = (d >= lo) & (d < lo + NH) & ((16 * j + iota) < valid)
            xv[pl.ds(16 * j, 16)] = jnp.where(keep, d - lo, NH)
        wait_gathers(b)
        issue_loads(i + 2, b)
        hr, ar = bufs[b][4], bufs[b][3]

        def group(g, carry):
            for u in range(4):
                t = 4 * g + u
                rows = 2 * t + half
                asv = plsc.load_gather(hr, [rows, tailcol])
                adv = plsc.load_gather(ar, [rows, lane8])
                al = asv + adv
                w = jnp.exp(jnp.maximum(al, 0.2 * al))
                for eo, off in ((0, 0), (1, 8)):
                    e = 2 * t + eo
                    for v in range(4):
                        hv = hr[e, pl.ds(16 * v, 16)]
                        wb = _dg(w, off + 2 * v + half)
                        hr[e, pl.ds(16 * v, 16)] = hv * wb
                plsc.store_scatter(hr, [rows, tailcol], w)
            return carry

        lax.fori_loop(0, C // 8, group, 0)
        pltpu.sync_copy(hr, accum.at[xv], add=True)

    pltpu.sync_copy(srce.at[pl.ds(s * EPS, C)], src0)
    pltpu.sync_copy(dste.at[pl.ds(s * EPS, C)], dst0)
    issue_gathers(0)
    issue_loads(1, 1)

    def pair(k, carry):
        phase(2 * k, 0)
        phase(2 * k + 1, 1)
        return carry

    lax.fori_loop(0, NCHUNK // 2, pair, 0)
    wait_gathers(0)
    wait_loads(1)
    plsc.subcore_barrier()
    pltpu.sync_copy(accum.at[pl.ds(s * STRIPE, STRIPE)],
                    accum_out.at[c, pl.ds(s * STRIPE, STRIPE)])


def _sc_b(srce, dste, h1t, ad1, init1):
    mesh = plsc.VectorSubcoreMesh(core_axis_name="c", subcore_axis_name="s",
                                  num_cores=2, num_subcores=16)
    return pl.kernel(
        _sc_b_body,
        out_type=jax.ShapeDtypeStruct((2, RP, 72), jnp.float32),
        mesh=mesh,
        compiler_params=pltpu.CompilerParams(needs_layout_passes=False, use_tc_tiling_on_sc=False),
        scratch_types=[
            pltpu.VMEM((C,), jnp.int32),
            pltpu.VMEM((C,), jnp.int32),
            pltpu.VMEM((C,), jnp.int32),
            pltpu.VMEM((C, 8), jnp.float32),
            pltpu.VMEM((C, 72), jnp.float32),
            pltpu.SemaphoreType.DMA,
            pltpu.SemaphoreType.DMA,
            pltpu.VMEM((C,), jnp.int32),
            pltpu.VMEM((C,), jnp.int32),
            pltpu.VMEM((C,), jnp.int32),
            pltpu.VMEM((C, 8), jnp.float32),
            pltpu.VMEM((C, 72), jnp.float32),
            pltpu.SemaphoreType.DMA,
            pltpu.SemaphoreType.DMA,
            pltpu.VMEM_SHARED((RP, 72), jnp.float32),
        ],
    )(srce, dste, h1t, ad1, init1)


def _tc_c_body(acc_ref, rep_ref, b1_ref, w2_ref, av2_ref,
               h2t_ref, ad2t_ref, init2_ref):
    acc = acc_ref[0]
    num = acc[:, 0:64]
    den = acc[:, 64:72]
    dexp = _dot(den, rep_ref[...])
    o1 = num / dexp + b1_ref[...][None, :]
    o1 = jnp.where(o1 > 0, o1, jnp.exp(o1) - 1.0)
    h2 = _dot(o1, w2_ref[...])
    asad = _dot(h2, av2_ref[...])
    a_s = asad[:, 0:1]
    a_d = asad[:, 1:2]
    al = a_s + a_d
    w = jnp.exp(jnp.maximum(al, 0.2 * al))
    one = jnp.ones((BLK, 1), jnp.float32)
    z14 = jnp.zeros((BLK, 14), jnp.float32)
    z15 = jnp.zeros((BLK, 15), jnp.float32)
    z7 = jnp.zeros((BLK, 7), jnp.float32)
    h2t_ref[...] = jnp.concatenate([h2, one, a_s, z14], axis=1)
    ad2t_ref[...] = jnp.concatenate([a_d, z7], axis=1)
    init2_ref[0] = jnp.concatenate([h2 * w, w, z15], axis=1)


def _tc_c(acc1, REP, b1, W2, av2):
    return pl.pallas_call(
        _tc_c_body,
        grid=(NBLK,),
        in_specs=[
            pl.BlockSpec((1, BLK, 72), lambda i: (i // BPC, i % BPC, 0)),
            pl.BlockSpec((8, 64), lambda i: (0, 0)),
            pl.BlockSpec((64,), lambda i: (0,)),
            pl.BlockSpec((64, 32), lambda i: (0, 0)),
            pl.BlockSpec((32, 2), lambda i: (0, 0)),
        ],
        out_specs=[
            pl.BlockSpec((BLK, 48), lambda i: (i, 0)),
            pl.BlockSpec((BLK, 8), lambda i: (i, 0)),
            pl.BlockSpec((1, BLK, 48), lambda i: (i // BPC, i % BPC, 0)),
        ],
        out_shape=[
            jax.ShapeDtypeStruct((N, 48), jnp.float32),
            jax.ShapeDtypeStruct((N, 8), jnp.float32),
            jax.ShapeDtypeStruct((2, RP, 48), jnp.float32),
        ],
    )(acc1, REP, b1, W2, av2)


def _sc_d_body(srce, dste, h2t, ad2t, init2, accum_out,
               src0, dst0, idx0, ad0, h0, sl0, sg0,
               src1, dst1, idx1, adb1, hb1, sl1, sg1, accum):
    c = lax.axis_index("c")
    s = lax.axis_index("s")
    lo = c * NH
    pltpu.sync_copy(init2.at[c, pl.ds(s * STRIPE, STRIPE)],
                    accum.at[pl.ds(s * STRIPE, STRIPE)])
    plsc.subcore_barrier()
    iota = lax.iota(jnp.int32, 16)
    col33 = iota * 0 + 33
    col0 = iota * 0
    bufs = ((src0, dst0, idx0, ad0, h0, sl0, sg0),
            (src1, dst1, idx1, adb1, hb1, sl1, sg1))

    def issue_loads(i, b):
        goff = s * EPS + i * C
        pltpu.async_copy(srce.at[pl.ds(goff, C)], bufs[b][0], bufs[b][5])
        pltpu.async_copy(dste.at[pl.ds(goff, C)], bufs[b][1], bufs[b][5])

    def wait_loads(b):
        pltpu.make_async_copy(srce.at[pl.ds(0, C)], bufs[b][0], bufs[b][5]).wait()
        pltpu.make_async_copy(dste.at[pl.ds(0, C)], bufs[b][1], bufs[b][5]).wait()

    def issue_gathers(b):
        pltpu.async_copy(ad2t.at[bufs[b][1]], bufs[b][3], bufs[b][6])
        pltpu.async_copy(h2t.at[bufs[b][0]], bufs[b][4], bufs[b][6])

    def wait_gathers(b):
        pltpu.make_async_copy(ad2t.at[pl.ds(0, C)], bufs[b][3], bufs[b][6]).wait()
        pltpu.make_async_copy(h2t.at[pl.ds(0, C)], bufs[b][4], bufs[b][6]).wait()

    def phase(i, b):
        o = 1 - b
        wait_loads(o)
        issue_gathers(o)
        dv, xv = bufs[b][1], bufs[b][2]
        valid = jnp.minimum(jnp.maximum(EPS - i * C, 0), C)
        for j in range(C // 16):
            d = dv[pl.ds(16 * j, 16)]
            keep = (d >= lo) & (d < lo + NH) & ((16 * j + iota) < valid)
            xv[pl.ds(16 * j, 16)] = jnp.where(keep, d - lo, NH)
        wait_gathers(b)
        issue_loads(i + 2, b)
        hr, ar = bufs[b][4], bufs[b][3]

        def group(g, carry):
            base = 16 * g
            rows = base + iota
            asv = plsc.load_gather(hr, [rows, col33])
            adv = plsc.load_gather(ar, [rows, col0])
            al = asv + adv
            w = jnp.exp(jnp.maximum(al, 0.2 * al))
            for j in range(16):
                e = base + j
                wb = _dg(w, col0 + j)
                for v in range(3):
                    hr[e, pl.ds(16 * v, 16)] = hr[e, pl.ds(16 * v, 16)] * wb
            return carry

        lax.fori_loop(0, C // 16, group, 0)
        pltpu.sync_copy(hr, accum.at[xv], add=True)

    pltpu.sync_copy(srce.at[pl.ds(s * EPS, C)], src0)
    pltpu.sync_copy(dste.at[pl.ds(s * EPS, C)], dst0)
    issue_gathers(0)
    issue_loads(1, 1)

    def pair(k, carry):
        phase(2 * k, 0)
        phase(2 * k + 1, 1)
        return carry

    lax.fori_loop(0, NCHUNK // 2, pair, 0)
    wait_gathers(0)
    wait_loads(1)
    plsc.subcore_barrier()
    pltpu.sync_copy(accum.at[pl.ds(s * STRIPE, STRIPE)],
                    accum_out.at[c, pl.ds(s * STRIPE, STRIPE)])


def _sc_d(srce, dste, h2t, ad2t, init2):
    mesh = plsc.VectorSubcoreMesh(core_axis_name="c", subcore_axis_name="s",
                                  num_cores=2, num_subcores=16)
    return pl.kernel(
        _sc_d_body,
        out_type=jax.ShapeDtypeStruct((2, RP, 48), jnp.float32),
        mesh=mesh,
        compiler_params=pltpu.CompilerParams(needs_layout_passes=False, use_tc_tiling_on_sc=False),
        scratch_types=[
            pltpu.VMEM((C,), jnp.int32),
            pltpu.VMEM((C,), jnp.int32),
            pltpu.VMEM((C,), jnp.int32),
            pltpu.VMEM((C, 8), jnp.float32),
            pltpu.VMEM((C, 48), jnp.float32),
            pltpu.SemaphoreType.DMA,
            pltpu.SemaphoreType.DMA,
            pltpu.VMEM((C,), jnp.int32),
            pltpu.VMEM((C,), jnp.int32),
            pltpu.VMEM((C,), jnp.int32),
            pltpu.VMEM((C, 8), jnp.float32),
            pltpu.VMEM((C, 48), jnp.float32),
            pltpu.SemaphoreType.DMA,
            pltpu.SemaphoreType.DMA,
            pltpu.VMEM_SHARED((RP, 48), jnp.float32),
        ],
    )(srce, dste, h2t, ad2t, init2)


def _tc_e_body(acc_ref, batch_ref, b2_ref, linw_ref, linb_ref, out_ref, sacc):
    b = pl.program_id(0)
    acc = acc_ref[0]
    o2 = acc[:, 0:32] / acc[:, 32:33] + b2_ref[...][None, :]
    bb = batch_ref[0]
    gids = lax.broadcasted_iota(jnp.int32, (G, BLK), 0)
    oh = (gids == bb).astype(jnp.float32)
    va = jnp.concatenate([o2, jnp.ones((BLK, 1), jnp.float32)], axis=1)
    part = _dot(oh, va, _HI)

    @pl.when(b == 0)
    def _():
        sacc[...] = part

    @pl.when(b > 0)
    def _():
        sacc[...] = sacc[...] + part

    @pl.when(b == NBLK - 1)
    def _():
        tot = sacc[...]
        pooled = tot[:, 0:32] / jnp.maximum(tot[:, 32:33], 1.0)
        out_ref[...] = _dot(pooled, linw_ref[...], _HI) + linb_ref[...][None, :]


def _tc_e(acc2, batch3, b2, linW, linb):
    return pl.pallas_call(
        _tc_e_body,
        grid=(NBLK,),
        in_specs=[
            pl.BlockSpec((1, BLK, 48), lambda i: (i // BPC, i % BPC, 0)),
            pl.BlockSpec((1, 1, BLK), lambda i: (i, 0, 0)),
            pl.BlockSpec((32,), lambda i: (0,)),
            pl.BlockSpec((32, 2), lambda i: (0, 0)),
            pl.BlockSpec((2,), lambda i: (0,)),
        ],
        out_specs=pl.BlockSpec((G, 2), lambda i: (0, 0)),
        out_shape=jax.ShapeDtypeStruct((G, 2), jnp.float32),
        scratch_shapes=[pltpu.VMEM((G, 33), jnp.float32)],
    )(acc2, batch3, b2, linW, linb)


def kernel(x, edge_index, batch, W1, att_src1, att_dst1, b1,
           W2, att_src2, att_dst2, b2, linW, linb):
    pad = jnp.zeros((EPAD - E,), jnp.int32)
    srce = jnp.concatenate([edge_index[0], pad])
    dste = jnp.concatenate([edge_index[1], pad])

    eye8 = jnp.eye(8, dtype=jnp.float32)
    AS = (att_src1[0][:, :, None] * eye8[:, None, :]).reshape(64, 8)
    AD = (att_dst1[0][:, :, None] * eye8[:, None, :]).reshape(64, 8)
    SAD = jnp.concatenate([AS, AD], axis=1)
    REP = (eye8[:, :, None] * jnp.ones((1, 1, 8), jnp.float32)).reshape(8, 64)
    av2 = jnp.concatenate([att_src2.reshape(32, 1), att_dst2.reshape(32, 1)],
                          axis=1)

    h1t, ad1, init1 = _tc_a(x, W1, SAD, REP)
    acc1 = _sc_b(srce, dste, h1t, ad1, init1)
    h2t, ad2t, init2 = _tc_c(acc1, REP, b1, W2, av2)
    acc2 = _sc_d(srce, dste, h2t, ad2t, init2)
    return _tc_e(acc2, batch.reshape(NBLK, 1, BLK), b2, linW, linb)
